# baseline (device time: 174614 ns/iter reference)
import jax
import jax.numpy as jnp
from jax import lax
from jax.experimental import pallas as pl
from jax.experimental.pallas import tpu as pltpu

N_DEV = 16
SQ = 1024
DH = 128
H_PER = 8
D_MODEL = 1024
CHUNK = SQ // N_DEV
SCALE = 0.08838834764831843
WINDOW = 128


def kernel(x, Wq, K_ext, V_ext, Wo):
    my = lax.axis_index("i")
    h0 = my * H_PER
    K_loc = jnp.transpose(lax.dynamic_slice_in_dim(K_ext[0], h0, H_PER, axis=1), (1, 0, 2))
    V_loc = jnp.transpose(lax.dynamic_slice_in_dim(V_ext[0], h0, H_PER, axis=1), (1, 0, 2))

    def body(x_ref, wq_ref, k_ref, v_ref, wo_ref, out_ref,
             ctx_ref, acc_ref, rs_buf, send_sems, recv_sems):
        my_i = lax.axis_index("i")
        left = lax.rem(my_i + N_DEV - 1, N_DEV)
        right = lax.rem(my_i + 1, N_DEV)

        barrier_sem = pltpu.get_barrier_semaphore()
        for nbr in (left, right):
            pl.semaphore_signal(
                barrier_sem, inc=1,
                device_id=(nbr,), device_id_type=pl.DeviceIdType.MESH,
            )
        pl.semaphore_wait(barrier_sem, 2)

        q = jnp.dot(x_ref[...], wq_ref[...], preferred_element_type=jnp.float32)

        qi = lax.broadcasted_iota(jnp.int32, (SQ, SQ), 0)
        ki = lax.broadcasted_iota(jnp.int32, (SQ, SQ), 1)
        mask = jnp.abs(qi - ki) <= WINDOW

        for h in range(H_PER):
            qh = q[:, h * DH:(h + 1) * DH]
            s = lax.dot_general(
                qh, k_ref[h],
                dimension_numbers=(((1,), (1,)), ((), ())),
                preferred_element_type=jnp.float32,
            ) * SCALE
            s = jnp.where(mask, s, -1e9)
            m = jnp.max(s, axis=1, keepdims=True)
            w = jnp.exp(s - m)
            w = w / jnp.sum(w, axis=1, keepdims=True)
            ctx_ref[:, h * DH:(h + 1) * DH] = jnp.dot(
                w, v_ref[h], preferred_element_type=jnp.float32)

        acc_ref[...] = jnp.dot(ctx_ref[...], wo_ref[...],
                               preferred_element_type=jnp.float32)

        for s_ in range(N_DEV - 1):
            send_c = lax.rem(my_i - s_ + N_DEV, N_DEV)
            recv_c = lax.rem(my_i - s_ - 1 + 2 * N_DEV, N_DEV)
            rdma = pltpu.make_async_remote_copy(
                src_ref=acc_ref.at[pl.ds(send_c * CHUNK, CHUNK), :],
                dst_ref=rs_buf.at[s_],
                send_sem=send_sems.at[s_],
                recv_sem=recv_sems.at[s_],
                device_id=(right,),
                device_id_type=pl.DeviceIdType.MESH,
            )
            rdma.start()
            rdma.wait()
            cur = acc_ref[pl.ds(recv_c * CHUNK, CHUNK), :]
            acc_ref[pl.ds(recv_c * CHUNK, CHUNK), :] = cur + rs_buf[s_]

        for s_ in range(N_DEV - 1):
            c = lax.rem(my_i + 1 - s_ + 2 * N_DEV, N_DEV)
            rdma = pltpu.make_async_remote_copy(
                src_ref=acc_ref.at[pl.ds(c * CHUNK, CHUNK), :],
                dst_ref=acc_ref.at[pl.ds(c * CHUNK, CHUNK), :],
                send_sem=send_sems.at[N_DEV - 1 + s_],
                recv_sem=recv_sems.at[N_DEV - 1 + s_],
                device_id=(right,),
                device_id_type=pl.DeviceIdType.MESH,
            )
            rdma.start()
            rdma.wait()

        out_ref[...] = acc_ref[...]

    out = pl.pallas_call(
        body,
        out_shape=jax.ShapeDtypeStruct((SQ, D_MODEL), jnp.float32),
        in_specs=[pl.BlockSpec(memory_space=pltpu.VMEM)] * 5,
        out_specs=pl.BlockSpec(memory_space=pltpu.VMEM),
        scratch_shapes=[
            pltpu.VMEM((SQ, D_MODEL), jnp.float32),
            pltpu.VMEM((SQ, D_MODEL), jnp.float32),
            pltpu.VMEM((N_DEV - 1, CHUNK, D_MODEL), jnp.float32),
            pltpu.SemaphoreType.DMA((2 * (N_DEV - 1),)),
            pltpu.SemaphoreType.DMA((2 * (N_DEV - 1),)),
        ],
        compiler_params=pltpu.CompilerParams(collective_id=0),
    )(x[0], Wq, K_loc, V_loc, Wo)
    return out[None, :, :]


# device time: 113828 ns/iter; 1.5340x vs baseline; 1.5340x over previous
import jax
import jax.numpy as jnp
from jax import lax
from jax.experimental import pallas as pl
from jax.experimental.pallas import tpu as pltpu

N_DEV = 16
SQ = 1024
DH = 128
H_PER = 8
D_MODEL = 1024
SCALE = 0.08838834764831843
WINDOW = 128
HALF = SQ // 2

C = 2
CW = D_MODEL // C

SIZES = [512, 256, 128, 64]
RS_ROW_OFF = [0, 512, 768, 896]


def kernel(x, Wq, K_ext, V_ext, Wo):
    my = lax.axis_index("i")
    h0 = my * H_PER
    K_loc = jnp.transpose(lax.dynamic_slice_in_dim(K_ext[0], h0, H_PER, axis=1), (1, 0, 2))
    V_loc = jnp.transpose(lax.dynamic_slice_in_dim(V_ext[0], h0, H_PER, axis=1), (1, 0, 2))

    def body(x_ref, wq_ref, k_ref, v_ref, wo_ref, out_ref,
             ctx_ref, acc_ref, rs_buf, send_sems, recv_sems):
        my_i = lax.axis_index("i")
        z = lax.div(my_i, 4)
        p = lax.rem(my_i, 4)

        x_bit = ((p == 1) | (p == 2)).astype(jnp.int32)
        y_bit = (p >= 2).astype(jnp.int32)
        z0_bit = lax.rem(z, 2)
        z1_bit = lax.div(z, 2)
        bits = [x_bit, y_bit, z0_bit, z1_bit]
        partners = [
            z * 4 + (p ^ 1),
            z * 4 + (3 - p),
            (z ^ 1) * 4 + p,
            (z ^ 2) * 4 + p,
        ]

        barrier_sem = pltpu.get_barrier_semaphore()
        for nbr in partners:
            pl.semaphore_signal(
                barrier_sem, inc=1,
                device_id=(nbr,), device_id_type=pl.DeviceIdType.MESH,
            )
        pl.semaphore_wait(barrier_sem, 4)

        send_off, keep_off = [], []
        b = jnp.int32(0)
        for s in range(4):
            send_off.append(b + (1 - bits[s]) * SIZES[s])
            b = b + bits[s] * SIZES[s]
            keep_off.append(b)
        v_off = [b]
        for j in range(4):
            v_off.append(v_off[j] - bits[3 - j] * (64 << j))

        def compute_half(off):
            xv = x_ref[pl.ds(off, HALF), :]
            q = jnp.dot(xv, wq_ref[...], preferred_element_type=jnp.float32)
            qi = lax.broadcasted_iota(jnp.int32, (HALF, SQ), 0) + off
            ki = lax.broadcasted_iota(jnp.int32, (HALF, SQ), 1)
            mask = jnp.abs(qi - ki) <= WINDOW
            for h in range(H_PER):
                qh = q[:, h * DH:(h + 1) * DH]
                sc = lax.dot_general(
                    qh, k_ref[h],
                    dimension_numbers=(((1,), (1,)), ((), ())),
                    preferred_element_type=jnp.float32,
                ) * SCALE
                sc = jnp.where(mask, sc, -1e9)
                m = jnp.max(sc, axis=1, keepdims=True)
                w = jnp.exp(sc - m)
                w = w / jnp.sum(w, axis=1, keepdims=True)
                ctx_ref[pl.ds(off, HALF), pl.ds(h * DH, DH)] = jnp.dot(
                    w, v_ref[h], preferred_element_type=jnp.float32)
            acc_ref[pl.ds(off, HALF), :] = jnp.dot(
                ctx_ref[pl.ds(off, HALF), :], wo_ref[...],
                preferred_element_type=jnp.float32)

        def mk_rs(s, c):
            return pltpu.make_async_remote_copy(
                src_ref=acc_ref.at[pl.ds(send_off[s], SIZES[s]), pl.ds(c * CW, CW)],
                dst_ref=rs_buf.at[pl.ds(RS_ROW_OFF[s], SIZES[s]), pl.ds(c * CW, CW)],
                send_sem=send_sems.at[s * C + c],
                recv_sem=recv_sems.at[s * C + c],
                device_id=(partners[s],),
                device_id_type=pl.DeviceIdType.MESH,
            )

        def mk_ag(j, c):
            sz = 64 << j
            blk = (pl.ds(v_off[j], sz), pl.ds(c * CW, CW))
            return pltpu.make_async_remote_copy(
                src_ref=acc_ref.at[blk],
                dst_ref=acc_ref.at[blk],
                send_sem=send_sems.at[4 * C + j * C + c],
                recv_sem=recv_sems.at[4 * C + j * C + c],
                device_id=(partners[3 - j],),
                device_id_type=pl.DeviceIdType.MESH,
            )

        compute_half(send_off[0])
        rd = {}
        for c in range(C):
            rd[(0, c)] = mk_rs(0, c)
            rd[(0, c)].start()
        compute_half(keep_off[0])

        for s in range(4):
            for c in range(C):
                rd[(s, c)].wait()
                rows = (pl.ds(keep_off[s], SIZES[s]), pl.ds(c * CW, CW))
                slot = (pl.ds(RS_ROW_OFF[s], SIZES[s]), pl.ds(c * CW, CW))
                acc_ref[rows] = acc_ref[rows] + rs_buf[slot]
                if s < 3:
                    rd[(s + 1, c)] = mk_rs(s + 1, c)
                    rd[(s + 1, c)].start()

        ag = {}
        for c in range(C):
            ag[(0, c)] = mk_ag(0, c)
            ag[(0, c)].start()
        for j in range(4):
            for c in range(C):
                ag[(j, c)].wait()
                if j < 3:
                    ag[(j + 1, c)] = mk_ag(j + 1, c)
                    ag[(j + 1, c)].start()

        out_ref[...] = acc_ref[...]

    out = pl.pallas_call(
        body,
        out_shape=jax.ShapeDtypeStruct((SQ, D_MODEL), jnp.float32),
        in_specs=[pl.BlockSpec(memory_space=pltpu.VMEM)] * 5,
        out_specs=pl.BlockSpec(memory_space=pltpu.VMEM),
        scratch_shapes=[
            pltpu.VMEM((SQ, D_MODEL), jnp.float32),
            pltpu.VMEM((SQ, D_MODEL), jnp.float32),
            pltpu.VMEM((960, D_MODEL), jnp.float32),
            pltpu.SemaphoreType.DMA((8 * C,)),
            pltpu.SemaphoreType.DMA((8 * C,)),
        ],
        compiler_params=pltpu.CompilerParams(collective_id=0),
    )(x[0], Wq, K_loc, V_loc, Wo)
    return out[None, :, :]
